# baseline (speedup 1.0000x reference)
import functools

import jax
import jax.numpy as jnp
from jax import lax
from jax.experimental import pallas as pl
from jax.experimental.pallas import tpu as pltpu
from jax.experimental.pallas import tpu_sc as plsc

LENGTH = 2048
N_PAIRS = 16

_mesh = plsc.ScalarSubcoreMesh(axis_name="c", num_cores=1)


@functools.partial(
    pl.kernel,
    mesh=_mesh,
    out_type=jax.ShapeDtypeStruct((N_PAIRS, 2), jnp.int32),
    scratch_types=[pltpu.SMEM((N_PAIRS, 2), jnp.int32)],
)
def _trunc_sc(idx_hbm, out_hbm, scratch):
    cid = lax.axis_index("c")

    @pl.when(cid == 0)
    def _():
        pltpu.sync_copy(idx_hbm, scratch)
        for i in range(N_PAIRS):
            b = scratch[i, 0]
            e = scratch[i, 1]
            scratch[i, 1] = jnp.minimum(b + LENGTH, e)
        pltpu.sync_copy(scratch, out_hbm)


_COPY_BLOCK = 2048
_HEAD_BLOCKS = 3


def _copy_head_body(x_ref, o_ref):
    o_ref[...] = x_ref[...]


def _tc_copy_head(values):
    rows, cols = values.shape
    return pl.pallas_call(
        _copy_head_body,
        grid=(_HEAD_BLOCKS,),
        in_specs=[pl.BlockSpec((_COPY_BLOCK, cols), lambda i: (i, 0))],
        out_specs=pl.BlockSpec((_COPY_BLOCK, cols), lambda i: (i, 0)),
        out_shape=jax.ShapeDtypeStruct(values.shape, values.dtype),
    )(values)


def _copy_tail_body(nblocks, _, x_ref, idx_ref, o_ref, idx_out_ref):
    o_ref[...] = x_ref[...]

    @pl.when(pl.program_id(0) == nblocks - 1)
    def _():
        idx_out_ref[...] = idx_ref[...]


def _tc_copy_tail(partial, values, idx):
    rows, cols = values.shape
    nblocks = rows // _COPY_BLOCK - _HEAD_BLOCKS
    return pl.pallas_call(
        functools.partial(_copy_tail_body, nblocks),
        grid=(nblocks,),
        in_specs=[
            pl.BlockSpec(memory_space=pl.ANY),
            pl.BlockSpec((_COPY_BLOCK, cols), lambda i: (i + _HEAD_BLOCKS, 0)),
            pl.BlockSpec((N_PAIRS, 2), lambda i: (0, 0)),
        ],
        out_specs=(
            pl.BlockSpec((_COPY_BLOCK, cols), lambda i: (i + _HEAD_BLOCKS, 0)),
            pl.BlockSpec((N_PAIRS, 2), lambda i: (0, 0)),
        ),
        out_shape=(
            jax.ShapeDtypeStruct(values.shape, values.dtype),
            jax.ShapeDtypeStruct((N_PAIRS, 2), jnp.int32),
        ),
        input_output_aliases={0: 0},
    )(partial, values, idx)


def kernel(values, indices):
    partial = _tc_copy_head(values)
    partial, indices = lax.optimization_barrier((partial, indices))
    trunc = _trunc_sc(indices)
    vals_out, idx_out = _tc_copy_tail(partial, values, trunc)
    return (vals_out, idx_out)
